# Initial kernel scaffold; baseline (speedup 1.0000x reference)
import jax
import jax.numpy as jnp
from jax.experimental import pallas as pl


def kernel(x, table):
    raise NotImplementedError("write your pallas kernel here")


# speedup vs baseline: 1.5756x; 1.5756x over previous
import functools

import numpy as np
import jax
import jax.numpy as jnp
from jax import lax
from jax.experimental import pallas as pl
from jax.experimental.pallas import tpu as pltpu
from jax.experimental.pallas import tpu_sc as plsc

EMBED = 128
LANES = 16
NC, NS = 2, 16
NW = NC * NS


def _positional_encoding_np(length: int, depth: int) -> np.ndarray:
    half = depth // 2
    positions = np.arange(length, dtype=np.float32)[:, None]
    depths = (np.arange(half, dtype=np.float32)[None, :] / float(half))
    angle_rates = (1.0 / (10000.0 ** depths)).astype(np.float32)
    angle_rads = positions * angle_rates
    return np.concatenate(
        [np.sin(angle_rads), np.cos(angle_rads)], axis=-1
    ).astype(np.float32)


@functools.cache
def _make_sc_kernel(b_total: int, seq: int, chunk: int):
    bpw = b_total // NW
    nchunks = bpw // chunk
    blocks_total = b_total // chunk
    mesh = plsc.VectorSubcoreMesh(
        core_axis_name="c", subcore_axis_name="s",
        num_cores=NC, num_subcores=NS,
    )

    @functools.partial(
        pl.kernel,
        out_type=jax.ShapeDtypeStruct((blocks_total, chunk, EMBED), jnp.float32),
        mesh=mesh,
        scratch_types=[
            pltpu.VMEM((nchunks, chunk), jnp.int32),
            pltpu.VMEM((chunk, EMBED), jnp.float32),
            pltpu.VMEM((seq, EMBED), jnp.float32),
            pltpu.SemaphoreType.DMA,
        ],
    )
    def sc_kernel(idx_hbm, table_hbm, pe_hbm, out_hbm, idx_v, rows_v, pe_v, sem):
        wid = lax.axis_index("s") * NC + lax.axis_index("c")
        pltpu.sync_copy(idx_hbm.at[wid], idx_v)
        pltpu.sync_copy(pe_hbm, pe_v)

        phases = seq // chunk

        def chunk_body(ci, _):
            pltpu.async_copy(table_hbm.at[idx_v.at[ci]], rows_v, sem).wait()
            pe_off = (ci % phases) * chunk

            def row_body(r, _):
                pr = pe_off + r
                for j in range(EMBED // LANES):
                    sl = pl.ds(j * LANES, LANES)
                    rows_v[r, sl] = rows_v[r, sl] + pe_v[pr, sl]
                return 0

            lax.fori_loop(0, chunk, row_body, 0, unroll=2)
            pltpu.sync_copy(rows_v, out_hbm.at[wid * nchunks + ci])
            return 0

        lax.fori_loop(0, nchunks, chunk_body, 0)

    return sc_kernel


def kernel(x, table):
    batch, seq = x.shape
    b_total = batch * seq
    chunk = 100
    idx = x.reshape(NW, (b_total // NW) // chunk, chunk).astype(jnp.int32)
    pe = jnp.asarray(_positional_encoding_np(seq, EMBED))
    sc = _make_sc_kernel(b_total, seq, chunk)
    out = sc(idx, table, pe)
    return out.reshape(batch, seq, EMBED)


# speedup vs baseline: 2.7376x; 1.7375x over previous
import functools

import numpy as np
import jax
import jax.numpy as jnp
from jax import lax
from jax.experimental import pallas as pl
from jax.experimental.pallas import tpu as pltpu
from jax.experimental.pallas import tpu_sc as plsc

EMBED = 128
LANES = 16
NC, NS = 2, 16
NW = NC * NS
NBUF = 4


def _positional_encoding_np(length: int, depth: int) -> np.ndarray:
    half = depth // 2
    positions = np.arange(length, dtype=np.float32)[:, None]
    depths = (np.arange(half, dtype=np.float32)[None, :] / float(half))
    angle_rates = (1.0 / (10000.0 ** depths)).astype(np.float32)
    angle_rads = positions * angle_rates
    return np.concatenate(
        [np.sin(angle_rads), np.cos(angle_rads)], axis=-1
    ).astype(np.float32)


@functools.cache
def _make_sc_kernel(b_total: int, seq: int, chunk: int):
    bpw = b_total // NW
    nchunks = bpw // chunk
    blocks_total = b_total // chunk
    assert nchunks % NBUF == 0 and seq % chunk == 0
    mesh = plsc.VectorSubcoreMesh(
        core_axis_name="c", subcore_axis_name="s",
        num_cores=NC, num_subcores=NS,
    )

    @functools.partial(
        pl.kernel,
        out_type=jax.ShapeDtypeStruct((blocks_total, chunk, EMBED), jnp.float32),
        mesh=mesh,
        scratch_types=[
            pltpu.VMEM((nchunks, chunk), jnp.int32),
            pltpu.VMEM((NBUF, chunk, EMBED), jnp.float32),
            pltpu.VMEM((seq, EMBED), jnp.float32),
        ] + [pltpu.SemaphoreType.DMA] * (2 * NBUF),
    )
    def sc_kernel(idx_hbm, table_hbm, pe_hbm, out_hbm, idx_v, rows_v, pe_v,
                  *sems):
        gsem = sems[:NBUF]
        ssem = sems[NBUF:]
        wid = lax.axis_index("s") * NC + lax.axis_index("c")
        pltpu.sync_copy(idx_hbm.at[wid], idx_v)
        pltpu.sync_copy(pe_hbm, pe_v)

        phases = seq // chunk

        for b in range(NBUF):
            pltpu.async_copy(table_hbm.at[idx_v.at[b]], rows_v.at[b], gsem[b])

        def add_pe(b, ci):
            pe_off = (ci % phases) * chunk

            def row_body(r, _):
                pr = pe_off + r
                for j in range(EMBED // LANES):
                    sl = pl.ds(j * LANES, LANES)
                    rows_v[b, r, sl] = rows_v[b, r, sl] + pe_v[pr, sl]
                return 0

            lax.fori_loop(0, chunk, row_body, 0, unroll=2)

        @pl.loop(0, nchunks, step=NBUF)
        def _(ci0):
            for b in range(NBUF):
                ci = ci0 + b
                pltpu.make_async_copy(
                    table_hbm.at[idx_v.at[ci]], rows_v.at[b], gsem[b]).wait()
                add_pe(b, ci)
                pltpu.async_copy(
                    rows_v.at[b], out_hbm.at[wid * nchunks + ci], ssem[b])
                cg = ci + NBUF - 1
                bb = (b - 1) % NBUF

                @pl.when(jnp.logical_and(cg >= NBUF, cg < nchunks))
                def _():
                    pltpu.make_async_copy(
                        rows_v.at[bb], out_hbm.at[wid * nchunks + ci - 1],
                        ssem[bb]).wait()
                    pltpu.async_copy(
                        table_hbm.at[idx_v.at[cg]], rows_v.at[bb], gsem[bb])

        for b in range(NBUF):
            ci = nchunks - NBUF + b
            pltpu.make_async_copy(
                rows_v.at[b], out_hbm.at[wid * nchunks + ci], ssem[b]).wait()

    return sc_kernel


def kernel(x, table):
    batch, seq = x.shape
    b_total = batch * seq
    chunk = 100
    idx = x.reshape(NW, (b_total // NW) // chunk, chunk).astype(jnp.int32)
    pe = jnp.asarray(_positional_encoding_np(seq, EMBED))
    sc = _make_sc_kernel(b_total, seq, chunk)
    out = sc(idx, table, pe)
    return out.reshape(batch, seq, EMBED)
